# baseline (speedup 1.0000x reference)
import jax
import jax.numpy as jnp
from jax import lax
from jax.experimental import pallas as pl
from jax.experimental.pallas import tpu as pltpu
from jax.experimental.pallas import tpu_sc as plsc

N = 10
E_PAD = 96
NPAD = 16


def _adj_sc_kernel(src_hbm, dst_hbm, a_hbm, src_v, dst_v, dis_v, a_v):
    cid = lax.axis_index("c")
    sid = lax.axis_index("s")

    @pl.when(jnp.logical_and(cid == 0, sid == 0))
    def _():
        pltpu.sync_copy(src_hbm, src_v)
        pltpu.sync_copy(dst_hbm, dst_v)
        zero = jnp.zeros((NPAD,), jnp.float32)
        dis_v[...] = zero
        for r in range(NPAD):
            a_v[r, :] = zero
        ones = jnp.ones((NPAD,), jnp.float32)
        for i in range(E_PAD // NPAD):
            d = dst_v[pl.ds(i * NPAD, NPAD)]
            plsc.addupdate_scatter(dis_v, [d], ones)
        deg = dis_v[...] + 1.0
        ib = plsc.bitcast(deg, jnp.int32)
        y = plsc.bitcast(jnp.full((NPAD,), 0x5F3759DF, jnp.int32) - (ib >> 1),
                         jnp.float32)
        for _ in range(4):
            y = y * (1.5 - 0.5 * deg * y * y)
        dis_v[...] = y
        for i in range(E_PAD // NPAD):
            s = src_v[pl.ds(i * NPAD, NPAD)]
            d = dst_v[pl.ds(i * NPAD, NPAD)]
            dis_s = plsc.load_gather(dis_v, [s])
            dis_d = plsc.load_gather(dis_v, [d])
            plsc.addupdate_scatter(a_v, [d, s], dis_s * dis_d)
        iot = lax.iota(jnp.int32, 16)
        plsc.addupdate_scatter(a_v, [iot, iot], y * y)
        pltpu.sync_copy(a_v, a_hbm)


def _build_adjacency(src, dst):
    mesh = plsc.VectorSubcoreMesh(core_axis_name="c", subcore_axis_name="s",
                                  num_cores=2)
    return pl.kernel(
        _adj_sc_kernel,
        mesh=mesh,
        compiler_params=pltpu.CompilerParams(needs_layout_passes=False),
        out_type=jax.ShapeDtypeStruct((NPAD, NPAD), jnp.float32),
        scratch_types=[
            pltpu.VMEM((E_PAD,), jnp.int32),
            pltpu.VMEM((E_PAD,), jnp.int32),
            pltpu.VMEM((NPAD,), jnp.float32),
            pltpu.VMEM((NPAD, NPAD), jnp.float32),
        ],
    )(src, dst)


def _xw1_kernel(x_ref, w1_ref, out_ref):
    out_ref[:, :] = jnp.dot(x_ref[:, :], w1_ref[:, :],
                            preferred_element_type=jnp.float32)


def _tail_kernel(a_ref, xw_ref, b1_ref, b2_ref, bc_ref, w2_hbm, wc_hbm,
                 out_ref, w2_v, wc_v, sem2, semc):
    f32 = jnp.float32
    cp2 = pltpu.make_async_copy(w2_hbm, w2_v, sem2)
    cpc = pltpu.make_async_copy(wc_hbm, wc_v, semc)
    cp2.start()
    cpc.start()
    A = a_ref[:N, :N]
    h1 = jnp.maximum(jnp.dot(A, xw_ref[:, :], preferred_element_type=f32)
                     + b1_ref[:, :], 0.0)
    cp2.wait()
    hw = jnp.dot(h1, w2_v[:, :], preferred_element_type=f32)
    h2 = jnp.maximum(jnp.dot(A, hw, preferred_element_type=f32)
                     + b2_ref[:, :], 0.0)
    cpc.wait()
    logits = bc_ref[:, :]
    for n in range(N):
        logits = logits + jnp.dot(h2[n:n + 1, :], wc_v[n],
                                  preferred_element_type=f32)
    m = jnp.max(logits, axis=1, keepdims=True)
    p = jnp.exp(logits - m)
    out_ref[:, :] = p / jnp.sum(p, axis=1, keepdims=True)


@jax.jit
def kernel(x, edge_index, W1, b1, W2, b2, Wc, bc):
    E = edge_index.shape[1]
    ei = edge_index.astype(jnp.int32)
    pad = jnp.full((2, E_PAD - E), 15, dtype=jnp.int32)
    ei = jnp.concatenate([ei, pad], axis=1)
    A = _build_adjacency(ei[0], ei[1])

    inf, hid = W1.shape
    ncls = Wc.shape[1]
    xw1 = pl.pallas_call(
        _xw1_kernel,
        out_shape=jax.ShapeDtypeStruct((N, hid), jnp.float32),
    )(x, W1)

    vmem = pl.BlockSpec(memory_space=pltpu.MemorySpace.VMEM)
    hbm = pl.BlockSpec(memory_space=pltpu.MemorySpace.HBM)
    out = pl.pallas_call(
        _tail_kernel,
        out_shape=jax.ShapeDtypeStruct((1, ncls), jnp.float32),
        in_specs=[vmem, vmem, vmem, vmem, vmem, hbm, hbm],
        out_specs=vmem,
        scratch_shapes=[
            pltpu.VMEM((hid, hid), jnp.float32),
            pltpu.VMEM((N, hid, ncls), jnp.float32),
            pltpu.SemaphoreType.DMA,
            pltpu.SemaphoreType.DMA,
        ],
    )(A, xw1, b1.reshape(1, hid), b2.reshape(1, hid), bc.reshape(1, ncls),
      W2, Wc.reshape(N, hid, ncls))
    return out


# speedup vs baseline: 1.0420x; 1.0420x over previous
import jax
import jax.numpy as jnp
from jax import lax
from jax.experimental import pallas as pl
from jax.experimental.pallas import tpu as pltpu
from jax.experimental.pallas import tpu_sc as plsc

N = 10
E_PAD = 96
NPAD = 16


def _adj_sc_kernel(src_hbm, dst_hbm, a_hbm, src_v, dst_v, dis_v, a_v):
    cid = lax.axis_index("c")
    sid = lax.axis_index("s")

    @pl.when(jnp.logical_and(cid == 0, sid == 0))
    def _():
        pltpu.sync_copy(src_hbm, src_v)
        pltpu.sync_copy(dst_hbm, dst_v)
        zero = jnp.zeros((NPAD,), jnp.float32)
        dis_v[...] = zero
        for r in range(NPAD):
            a_v[r, :] = zero
        ones = jnp.ones((NPAD,), jnp.float32)
        for i in range(E_PAD // NPAD):
            d = dst_v[pl.ds(i * NPAD, NPAD)]
            plsc.addupdate_scatter(dis_v, [d], ones)
        deg = dis_v[...] + 1.0
        ib = plsc.bitcast(deg, jnp.int32)
        y = plsc.bitcast(jnp.full((NPAD,), 0x5F3759DF, jnp.int32) - (ib >> 1),
                         jnp.float32)
        for _ in range(4):
            y = y * (1.5 - 0.5 * deg * y * y)
        dis_v[...] = y
        for i in range(E_PAD // NPAD):
            s = src_v[pl.ds(i * NPAD, NPAD)]
            d = dst_v[pl.ds(i * NPAD, NPAD)]
            dis_s = plsc.load_gather(dis_v, [s])
            dis_d = plsc.load_gather(dis_v, [d])
            plsc.addupdate_scatter(a_v, [d, s], dis_s * dis_d)
        iot = lax.iota(jnp.int32, 16)
        plsc.addupdate_scatter(a_v, [iot, iot], y * y)
        pltpu.sync_copy(a_v, a_hbm)


def _build_adjacency(src, dst):
    mesh = plsc.VectorSubcoreMesh(core_axis_name="c", subcore_axis_name="s",
                                  num_cores=2)
    return pl.kernel(
        _adj_sc_kernel,
        mesh=mesh,
        compiler_params=pltpu.CompilerParams(needs_layout_passes=False,
                                             skip_device_barrier=True),
        out_type=jax.ShapeDtypeStruct((NPAD, NPAD), jnp.float32),
        scratch_types=[
            pltpu.VMEM((E_PAD,), jnp.int32),
            pltpu.VMEM((E_PAD,), jnp.int32),
            pltpu.VMEM((NPAD,), jnp.float32),
            pltpu.VMEM((NPAD, NPAD), jnp.float32),
        ],
    )(src, dst)


def _dense_kernel(a_ref, x_ref, b1_ref, b2_ref, bc_ref,
                  w1_hbm, w2_hbm, wc_hbm, out_ref,
                  w1_v, w2_v, wc_v, sem1, sem2, semc):
    f32 = jnp.float32
    cp1 = pltpu.make_async_copy(w1_hbm, w1_v, sem1)
    cp2 = pltpu.make_async_copy(w2_hbm, w2_v, sem2)
    cpc = pltpu.make_async_copy(wc_hbm, wc_v, semc)
    cp1.start()
    cp2.start()
    cpc.start()
    A = a_ref[:N, :N]
    cp1.wait()
    xw = jnp.dot(x_ref[:, :], w1_v[:, :], preferred_element_type=f32)
    h1 = jnp.maximum(jnp.dot(A, xw, preferred_element_type=f32)
                     + b1_ref[:, :], 0.0)
    cp2.wait()
    hw = jnp.dot(h1, w2_v[:, :], preferred_element_type=f32)
    h2 = jnp.maximum(jnp.dot(A, hw, preferred_element_type=f32)
                     + b2_ref[:, :], 0.0)
    cpc.wait()
    logits = bc_ref[:, :]
    for n in range(N):
        logits = logits + jnp.dot(h2[n:n + 1, :], wc_v[n],
                                  preferred_element_type=f32)
    m = jnp.max(logits, axis=1, keepdims=True)
    p = jnp.exp(logits - m)
    out_ref[:, :] = p / jnp.sum(p, axis=1, keepdims=True)


@jax.jit
def kernel(x, edge_index, W1, b1, W2, b2, Wc, bc):
    E = edge_index.shape[1]
    ei = edge_index.astype(jnp.int32)
    pad = jnp.full((2, E_PAD - E), 15, dtype=jnp.int32)
    ei = jnp.concatenate([ei, pad], axis=1)
    A = _build_adjacency(ei[0], ei[1])

    inf, hid = W1.shape
    ncls = Wc.shape[1]
    vmem = pl.BlockSpec(memory_space=pltpu.MemorySpace.VMEM)
    hbm = pl.BlockSpec(memory_space=pltpu.MemorySpace.HBM)
    out = pl.pallas_call(
        _dense_kernel,
        out_shape=jax.ShapeDtypeStruct((1, ncls), jnp.float32),
        in_specs=[vmem, vmem, vmem, vmem, vmem, hbm, hbm, hbm],
        out_specs=vmem,
        scratch_shapes=[
            pltpu.VMEM((inf, hid), jnp.float32),
            pltpu.VMEM((hid, hid), jnp.float32),
            pltpu.VMEM((N, hid, ncls), jnp.float32),
            pltpu.SemaphoreType.DMA,
            pltpu.SemaphoreType.DMA,
            pltpu.SemaphoreType.DMA,
        ],
    )(A, x, b1.reshape(1, hid), b2.reshape(1, hid), bc.reshape(1, ncls),
      W1, W2, Wc.reshape(N, hid, ncls))
    return out


# speedup vs baseline: 3.6339x; 3.4874x over previous
import jax
import jax.numpy as jnp
from jax.experimental import pallas as pl

N = 10
E_PAD = 96


def _fused_kernel(ei_ref, x_ref, w1_ref, b1_ref, w2_ref, b2_ref, wct_ref,
                  bc_ref, out_ref):
    f32 = jnp.float32
    src = ei_ref[0:1, :]
    dst = ei_ref[1:2, :]

    node_col = jax.lax.broadcasted_iota(jnp.int32, (N, E_PAD), 0)
    St = (src == node_col).astype(f32)
    Dt = (dst == node_col).astype(f32)

    deg = 1.0 + jnp.sum(Dt, axis=1, keepdims=True)
    dis = jax.lax.rsqrt(deg)

    dis_src = jnp.sum(St * dis, axis=0, keepdims=True)
    dis_dst = jnp.sum(Dt * dis, axis=0, keepdims=True)
    norm = dis_src * dis_dst

    A = jax.lax.dot_general(Dt * norm, St, (((1,), (1,)), ((), ())),
                            preferred_element_type=f32)
    eye = (jax.lax.broadcasted_iota(jnp.int32, (N, N), 0)
           == jax.lax.broadcasted_iota(jnp.int32, (N, N), 1)).astype(f32)
    A = A + eye * (1.0 / deg)

    xw = jnp.dot(x_ref[:, :], w1_ref[:, :], preferred_element_type=f32)
    h1 = jnp.maximum(jnp.dot(A, xw, preferred_element_type=f32)
                     + b1_ref[:, :], 0.0)
    hw = jnp.dot(h1, w2_ref[:, :], preferred_element_type=f32)
    h2 = jnp.maximum(jnp.dot(A, hw, preferred_element_type=f32)
                     + b2_ref[:, :], 0.0)

    parts = [jnp.sum(h2 * wct_ref[c]).reshape(1, 1)
             for c in range(wct_ref.shape[0])]
    logits = bc_ref[:, :] + jnp.concatenate(parts, axis=1)

    m = jnp.max(logits, axis=1, keepdims=True)
    p = jnp.exp(logits - m)
    out_ref[:, :] = p / jnp.sum(p, axis=1, keepdims=True)


@jax.jit
def kernel(x, edge_index, W1, b1, W2, b2, Wc, bc):
    E = edge_index.shape[1]
    ei = edge_index.astype(jnp.int32)
    pad = jnp.full((2, E_PAD - E), -1, dtype=jnp.int32)
    ei = jnp.concatenate([ei, pad], axis=1)
    hid = W1.shape[1]
    ncls = Wc.shape[1]
    wct = jnp.transpose(Wc).reshape(ncls, N, hid)
    out = pl.pallas_call(
        _fused_kernel,
        out_shape=jax.ShapeDtypeStruct((1, ncls), jnp.float32),
    )(ei, x, W1, b1.reshape(1, hid), W2, b2.reshape(1, hid), wct,
      bc.reshape(1, ncls))
    return out


# speedup vs baseline: 3.7859x; 1.0418x over previous
import jax
import jax.numpy as jnp
from jax.experimental import pallas as pl
from jax.experimental.pallas import tpu as pltpu

N = 10
E_PAD = 96


def _fused_kernel(ei_hbm, x_hbm, w1_hbm, b1_hbm, w2_hbm, b2_hbm, wct_hbm,
                  bc_hbm, out_ref,
                  ei_v, x_v, w1_v, b1_v, w2_v, b2_v, wct_v, bc_v,
                  s_ei, s_x, s_w1, s_b1, s_w2, s_b2, s_wct, s_bc):
    f32 = jnp.float32
    cps = [pltpu.make_async_copy(ei_hbm, ei_v, s_ei),
           pltpu.make_async_copy(x_hbm, x_v, s_x),
           pltpu.make_async_copy(w1_hbm, w1_v, s_w1),
           pltpu.make_async_copy(b1_hbm, b1_v, s_b1),
           pltpu.make_async_copy(w2_hbm, w2_v, s_w2),
           pltpu.make_async_copy(b2_hbm, b2_v, s_b2),
           pltpu.make_async_copy(wct_hbm, wct_v, s_wct),
           pltpu.make_async_copy(bc_hbm, bc_v, s_bc)]
    for cp in cps:
        cp.start()
    cp_ei, cp_x, cp_w1, cp_b1, cp_w2, cp_b2, cp_wct, cp_bc = cps

    cp_ei.wait()
    src = ei_v[0:1, :]
    dst = ei_v[1:2, :]
    node_col = jax.lax.broadcasted_iota(jnp.int32, (N, E_PAD), 0)
    St = (src == node_col).astype(f32)
    Dt = (dst == node_col).astype(f32)

    deg = 1.0 + jnp.sum(Dt, axis=1, keepdims=True)
    dis = jax.lax.rsqrt(deg)
    dis_src = jnp.sum(St * dis, axis=0, keepdims=True)
    dis_dst = jnp.sum(Dt * dis, axis=0, keepdims=True)
    norm = dis_src * dis_dst

    A = jax.lax.dot_general(Dt * norm, St, (((1,), (1,)), ((), ())),
                            preferred_element_type=f32)
    eye = (jax.lax.broadcasted_iota(jnp.int32, (N, N), 0)
           == jax.lax.broadcasted_iota(jnp.int32, (N, N), 1)).astype(f32)
    A = A + eye * (1.0 / deg)

    cp_x.wait()
    cp_w1.wait()
    xw = jnp.dot(x_v[:, :], w1_v[:, :], preferred_element_type=f32)
    cp_b1.wait()
    h1 = jnp.maximum(jnp.dot(A, xw, preferred_element_type=f32)
                     + b1_v[:, :], 0.0)
    cp_w2.wait()
    hw = jnp.dot(h1, w2_v[:, :], preferred_element_type=f32)
    cp_b2.wait()
    h2 = jnp.maximum(jnp.dot(A, hw, preferred_element_type=f32)
                     + b2_v[:, :], 0.0)

    cp_wct.wait()
    parts = [jnp.sum(h2 * wct_v[c]).reshape(1, 1)
             for c in range(wct_v.shape[0])]
    cp_bc.wait()
    logits = bc_v[:, :] + jnp.concatenate(parts, axis=1)

    m = jnp.max(logits, axis=1, keepdims=True)
    p = jnp.exp(logits - m)
    out_ref[:, :] = p / jnp.sum(p, axis=1, keepdims=True)


@jax.jit
def kernel(x, edge_index, W1, b1, W2, b2, Wc, bc):
    E = edge_index.shape[1]
    ei = edge_index.astype(jnp.int32)
    pad = jnp.full((2, E_PAD - E), -1, dtype=jnp.int32)
    ei = jnp.concatenate([ei, pad], axis=1)
    inf, hid = W1.shape
    ncls = Wc.shape[1]
    wct = jnp.transpose(Wc).reshape(ncls, N, hid)
    vmem = pl.BlockSpec(memory_space=pltpu.MemorySpace.VMEM)
    hbm = pl.BlockSpec(memory_space=pltpu.MemorySpace.HBM)
    out = pl.pallas_call(
        _fused_kernel,
        out_shape=jax.ShapeDtypeStruct((1, ncls), jnp.float32),
        in_specs=[hbm] * 8,
        out_specs=vmem,
        scratch_shapes=[
            pltpu.VMEM((2, E_PAD), jnp.int32),
            pltpu.VMEM((N, inf), jnp.float32),
            pltpu.VMEM((inf, hid), jnp.float32),
            pltpu.VMEM((1, hid), jnp.float32),
            pltpu.VMEM((hid, hid), jnp.float32),
            pltpu.VMEM((1, hid), jnp.float32),
            pltpu.VMEM((ncls, N, hid), jnp.float32),
            pltpu.VMEM((1, ncls), jnp.float32),
        ] + [pltpu.SemaphoreType.DMA] * 8,
    )(ei, x, W1, b1.reshape(1, hid), W2, b2.reshape(1, hid), wct,
      bc.reshape(1, ncls))
    return out
